# Initial kernel scaffold; baseline (speedup 1.0000x reference)
import jax
import jax.numpy as jnp
from jax.experimental import pallas as pl


def kernel(x, weight, bias):
    raise NotImplementedError("write your pallas kernel here")


# speedup vs baseline: 1.9986x; 1.9986x over previous
import jax
import jax.numpy as jnp
from jax import lax
from jax.experimental import pallas as pl
from jax.experimental.pallas import tpu as pltpu


def _linear_kernel(x_ref, w_ref, b_ref, o_ref):
    o_ref[...] = (
        lax.dot_general(
            x_ref[...],
            w_ref[...],
            dimension_numbers=(((1,), (1,)), ((), ())),
            preferred_element_type=jnp.float32,
        )
        + b_ref[...]
    )


@jax.jit
def _linear(x, weight, bias):
    M, K = x.shape
    N, Kw = weight.shape
    assert K == Kw, "weight inner dim must match x"

    xb = x.astype(jnp.bfloat16)
    wb = weight.astype(jnp.bfloat16)
    b2d = bias.reshape(1, N).astype(jnp.float32)

    bm = 1024 if M % 1024 == 0 else M
    bn = 1024 if N % 1024 == 0 else N
    grid = (M // bm, N // bn)

    cost = pl.CostEstimate(
        flops=2 * M * N * K,
        transcendentals=0,
        bytes_accessed=2 * (M * K * (N // bn) + N * K * (M // bm)) + 4 * (M * N + N),
    )

    out = pl.pallas_call(
        _linear_kernel,
        out_shape=jax.ShapeDtypeStruct((M, N), jnp.float32),
        grid=grid,
        in_specs=[
            pl.BlockSpec((bm, K), lambda i, j: (i, 0)),
            pl.BlockSpec((bn, K), lambda i, j: (j, 0)),
            pl.BlockSpec((1, bn), lambda i, j: (0, j)),
        ],
        out_specs=pl.BlockSpec((bm, bn), lambda i, j: (i, j)),
        compiler_params=pltpu.CompilerParams(
            dimension_semantics=("parallel", "parallel"),
            vmem_limit_bytes=64 * 1024 * 1024,
        ),
        cost_estimate=cost,
    )(xb, wb, b2d)
    return out.astype(x.dtype)


def kernel(x, weight, bias):
    return _linear(x, weight, bias)
